# baseline (speedup 1.0000x reference)
import functools

import jax
import jax.numpy as jnp
from jax import lax
from jax.experimental import pallas as pl
from jax.experimental.pallas import tpu as pltpu
from jax.experimental.pallas import tpu_sc as plsc

N = 10000
E = 320000
NC = 2
NS = 16
NW = NC * NS
C = 32
T = 315
EP = T * NW * C
NP = 10240
RS = NP // NS
DR = NP // 128
TW = 128
BR = 1024


def _sc_edge_kernel(do):
    mesh = plsc.VectorSubcoreMesh(core_axis_name="c", subcore_axis_name="s")

    @functools.partial(
        pl.kernel,
        out_type=(
            jax.ShapeDtypeStruct((NC, NP, TW), jnp.float32),
            jax.ShapeDtypeStruct((NC, DR, 128), jnp.float32),
        ),
        mesh=mesh,
        compiler_params=pltpu.CompilerParams(needs_layout_passes=False),
        scratch_types=[
            pltpu.VMEM((C,), jnp.int32),
            pltpu.VMEM((C,), jnp.int32),
            pltpu.VMEM((C,), jnp.int32),
            pltpu.VMEM((C,), jnp.int32),
            pltpu.VMEM((C,), jnp.int32),
            pltpu.VMEM((C,), jnp.int32),
            pltpu.VMEM((C, TW), jnp.float32),
            pltpu.VMEM((C, TW), jnp.float32),
            pltpu.VMEM((C, TW), jnp.float32),
            pltpu.VMEM((C, TW), jnp.float32),
            pltpu.VMEM((C, TW), jnp.float32),
            pltpu.VMEM((C, TW), jnp.float32),
            pltpu.VMEM((C, TW), jnp.float32),
            pltpu.VMEM((C, TW), jnp.float32),
            pltpu.VMEM((C, TW), jnp.float32),
            pltpu.VMEM((do,), jnp.float32),
            pltpu.VMEM((DR, 128), jnp.float32),
            pltpu.VMEM((DR,), jnp.int32),
            pltpu.VMEM_SHARED((NP, TW), jnp.float32),
            pltpu.VMEM_SHARED((DR, 128), jnp.float32),
            pltpu.SemaphoreType.DMA,
            pltpu.SemaphoreType.DMA,
            pltpu.SemaphoreType.DMA,
            pltpu.SemaphoreType.DMA,
            pltpu.SemaphoreType.DMA,
            pltpu.SemaphoreType.DMA,
        ],
    )
    def edge_kernel(a_hbm, b_hbm, xl_hbm, src_hbm, dst_hbm, att_hbm, z_hbm,
                    num_out, den_out,
                    is0, is1, is2, id0, id1, id2, ra0, ra1, ra2,
                    rb0, rb1, rb2, rx0, rx1, rx2, att_v, den_local, row_ids,
                    num_sh, den_sh, g0, g1, g2, s0, s1, s2):
        idx_src = [is0, is1, is2]
        idx_dst = [id0, id1, id2]
        rows_a = [ra0, ra1, ra2]
        rows_b = [rb0, rb1, rb2]
        rows_xl = [rx0, rx1, rx2]
        gsem = [g0, g1, g2]
        ssem = [s0, s1, s2]

        c = lax.axis_index("c")
        s = lax.axis_index("s")
        wid = s * NC + c

        pltpu.sync_copy(z_hbm.at[pl.ds(s * RS, RS)],
                        num_sh.at[pl.ds(s * RS, RS)])

        @pl.when(s < 10)
        def _():
            pltpu.sync_copy(z_hbm.at[pl.ds(s * 8, 8)],
                            den_sh.at[pl.ds(s * 8, 8)])

        pltpu.sync_copy(att_hbm, att_v)

        def zden_body(r, carry):
            for k in range(8):
                den_local[r, pl.ds(k * 16, 16)] = jnp.zeros((16,), jnp.float32)
            return carry

        lax.fori_loop(0, DR, zden_body, 0)

        def rid_body(j, carry):
            row_ids[pl.ds(j * 16, 16)] = lax.iota(jnp.int32, 16) + j * 16
            return carry

        lax.fori_loop(0, DR // 16, rid_body, 0)
        plsc.subcore_barrier()

        def fire(a, b):
            base = (wid + a * NW) * C
            pltpu.sync_copy(src_hbm.at[pl.ds(base, C)], idx_src[b])
            pltpu.sync_copy(dst_hbm.at[pl.ds(base, C)], idx_dst[b])
            pltpu.async_copy(a_hbm.at[idx_src[b]], rows_a[b], gsem[b])
            pltpu.async_copy(b_hbm.at[idx_dst[b]], rows_b[b], gsem[b])
            pltpu.async_copy(xl_hbm.at[idx_src[b]], rows_xl[b], gsem[b])

        def wait_gathers(b):
            pltpu.make_async_copy(
                a_hbm.at[idx_src[b]], rows_a[b], gsem[b]).wait()
            pltpu.make_async_copy(
                b_hbm.at[idx_dst[b]], rows_b[b], gsem[b]).wait()
            pltpu.make_async_copy(
                xl_hbm.at[idx_src[b]], rows_xl[b], gsem[b]).wait()

        def fire_scatter(b):
            pltpu.async_copy(rows_xl[b], num_sh.at[idx_dst[b]],
                             ssem[b], add=True)

        def wait_scatter(b):
            pltpu.make_async_copy(
                rows_xl[b], num_sh.at[idx_dst[b]], ssem[b]).wait()

        def compute(b):
            def group_body(g, carry2):
                ev = lax.iota(jnp.int32, 16) + g * 16

                def alpha_blk(k16, acc):
                    base = k16 * 16
                    for dk in range(16):
                        kv = jnp.broadcast_to(base + dk, (16,))
                        m = (plsc.load_gather(rows_a[b], [ev, kv])
                             + plsc.load_gather(rows_b[b], [ev, kv]))
                        lr = jnp.maximum(m, 0.2 * m)
                        acc = acc + lr * plsc.load_gather(att_v, [kv])
                    return acc

                acc = lax.fori_loop(0, do // 16, alpha_blk,
                                    jnp.zeros((16,), jnp.float32))
                exv = jnp.exp(acc)

                def scale_blk(k16, carry3):
                    base = k16 * 16
                    for dk in range(16):
                        kv = jnp.broadcast_to(base + dk, (16,))
                        v = plsc.load_gather(rows_xl[b], [ev, kv])
                        plsc.store_scatter(rows_xl[b], [ev, kv], v * exv)
                    return carry3

                lax.fori_loop(0, do // 16, scale_blk, 0)
                dstv = idx_dst[b][pl.ds(g * 16, 16)]
                plsc.addupdate_scatter(den_local, [dstv >> 7, dstv & 127], exv)
                return carry2

            lax.fori_loop(0, C // 16, group_body, 0)

        fire(0, 0)
        fire(1, 1)

        def pipe_body(t, carry):
            for b in range(3):
                a = t * 3 + b
                wait_gathers(b)
                nb = (b + 2) % 3

                @pl.when(a + 2 < T)
                def _():
                    @pl.when(a >= 1)
                    def _():
                        wait_scatter(nb)
                    fire(a + 2, nb)

                compute(b)
                fire_scatter(b)
            return carry

        lax.fori_loop(0, T // 3, pipe_body, 0)
        for b in range(3):
            wait_scatter(b)
        pltpu.sync_copy(den_local, den_sh.at[row_ids], add=True)
        plsc.subcore_barrier()

        pltpu.sync_copy(num_sh.at[pl.ds(s * RS, RS)],
                        num_out.at[c, pl.ds(s * RS, RS)])

        @pl.when(s < 10)
        def _():
            pltpu.sync_copy(den_sh.at[pl.ds(s * 8, 8)],
                            den_out.at[c, pl.ds(s * 8, 8)])

    return edge_kernel


_sc_edge_128 = _sc_edge_kernel(128)
_sc_edge_64 = _sc_edge_kernel(64)


def _tc_layer(first):
    def body(*refs):
        if first:
            (h_ref, pos_ref, wlh_ref, wap_ref, wrh_ref, wbp_ref,
             xle_ref, a_ref, b_ref) = refs
            h = h_ref[...]
        else:
            (np_ref, dp_ref, bias_ref, pos_ref, wlh_ref, wap_ref, wrh_ref,
             wbp_ref, xle_ref, a_ref, b_ref) = refs
            num = np_ref[0] + np_ref[1]
            den = dp_ref[0] + dp_ref[1]
            h = num / (den[:, None] + 1e-16) + bias_ref[...]
            h = jnp.where(h > 0, h, jnp.exp(jnp.minimum(h, 0.0)) - 1.0)
        p = pos_ref[...]
        xl = (jnp.dot(h, wlh_ref[...], preferred_element_type=jnp.float32)
              + jnp.dot(p, wap_ref[0], preferred_element_type=jnp.float32))
        pa = jnp.dot(p, wap_ref[1], preferred_element_type=jnp.float32)
        xr = (jnp.dot(h, wrh_ref[...], preferred_element_type=jnp.float32)
              + jnp.dot(p, wbp_ref[0], preferred_element_type=jnp.float32))
        xle_ref[...] = xl
        a_ref[...] = xl - pa
        b_ref[...] = xr

    return body


def _run_tc_layer(h_or_numP, denF, bias, pos, Wlh, Wap2, Wrh, Wbp2, first):
    grid = (NP // BR,)
    full = lambda shape: pl.BlockSpec(shape, lambda i: (0,) * len(shape))
    w_specs = [full((TW, TW)), full((2, 3, TW)), full((TW, TW)),
               full((1, 3, TW))]
    row = lambda w: pl.BlockSpec((BR, w), lambda i: (i, 0))
    if first:
        in_specs = [row(TW), row(3)] + w_specs
        args = (h_or_numP, pos, Wlh, Wap2, Wrh, Wbp2)
    else:
        in_specs = [
            pl.BlockSpec((NC, BR, TW), lambda i: (0, i, 0)),
            pl.BlockSpec((NC, BR), lambda i: (0, i)),
            full((TW,)),
            row(3),
        ] + w_specs
        args = (h_or_numP, denF, bias, pos, Wlh, Wap2, Wrh, Wbp2)
    out_shapes = [jax.ShapeDtypeStruct((NP, TW), jnp.float32)] * 3
    out_specs = [row(TW)] * 3
    return pl.pallas_call(
        _tc_layer(first),
        grid=grid,
        in_specs=in_specs,
        out_specs=out_specs,
        out_shape=out_shapes,
    )(*args)


def _tc_final_body(np_ref, dp_ref, bias_ref, out_ref):
    num = np_ref[0] + np_ref[1]
    den = dp_ref[0] + dp_ref[1]
    h = num[:, :64] / (den[:, None] + 1e-16) + bias_ref[...]
    out_ref[...] = jnp.where(h > 0, h, jnp.exp(jnp.minimum(h, 0.0)) - 1.0)


def _run_tc_final(numP, denF, bias):
    return pl.pallas_call(
        _tc_final_body,
        grid=(NP // BR,),
        in_specs=[
            pl.BlockSpec((NC, BR, TW), lambda i: (0, i, 0)),
            pl.BlockSpec((NC, BR), lambda i: (0, i)),
            pl.BlockSpec((64,), lambda i: (0,)),
        ],
        out_specs=pl.BlockSpec((BR, 64), lambda i: (i, 0)),
        out_shape=jax.ShapeDtypeStruct((NP, 64), jnp.float32),
    )(numP, denF, bias)


def _prep(Wl, Wr, We, do):
    Wlh, Wlp = Wl[:TW], Wl[TW:]
    Wrh, Wrp = Wr[:TW], Wr[TW:]
    if do < TW:
        pad = [(0, 0), (0, TW - do)]
        Wlh, Wlp = jnp.pad(Wlh, pad), jnp.pad(Wlp, pad)
        Wrh, Wrp = jnp.pad(Wrh, pad), jnp.pad(Wrp, pad)
        We = jnp.pad(We, pad)
    Wap2 = jnp.stack([Wlp, We])
    Wbp2 = (Wrp + We)[None]
    return Wlh, Wap2, Wrh, Wbp2


def kernel(x, edge_index, pos, Wl0, Wr0, We0, att0, b0, Wl1, Wr1, We1, att1,
           b1, Wl2, Wr2, We2, att2, b2):
    src = jnp.pad(edge_index[0], (0, EP - E), constant_values=N)
    dst = jnp.pad(edge_index[1], (0, EP - E), constant_values=N)
    xp = jnp.pad(x, [(0, NP - N), (0, 0)])
    posp = jnp.pad(pos, [(0, NP - N), (0, 0)])
    z = jnp.zeros((NP, TW), jnp.float32)

    Wlh, Wap2, Wrh, Wbp2 = _prep(Wl0, Wr0, We0, 128)
    xle, a, b = _run_tc_layer(xp, None, None, posp, Wlh, Wap2, Wrh, Wbp2, True)
    numP, denP = _sc_edge_128(a, b, xle, src, dst, att0, z)

    Wlh, Wap2, Wrh, Wbp2 = _prep(Wl1, Wr1, We1, 128)
    xle, a, b = _run_tc_layer(numP, denP.reshape(NC, NP), b0, posp, Wlh, Wap2,
                              Wrh, Wbp2, False)
    numP, denP = _sc_edge_128(a, b, xle, src, dst, att1, z)

    Wlh, Wap2, Wrh, Wbp2 = _prep(Wl2, Wr2, We2, 64)
    xle, a, b = _run_tc_layer(numP, denP.reshape(NC, NP), b1, posp, Wlh, Wap2,
                              Wrh, Wbp2, False)
    numP, denP = _sc_edge_64(a, b, xle, src, dst, att2, z)

    out = _run_tc_final(numP, denP.reshape(NC, NP), b2)
    return out[:N]


# speedup vs baseline: 2.9195x; 2.9195x over previous
import functools

import jax
import jax.numpy as jnp
from jax import lax
from jax.experimental import pallas as pl
from jax.experimental.pallas import tpu as pltpu
from jax.experimental.pallas import tpu_sc as plsc

N = 10000
E = 320000
NC = 2
NS = 16
NW = NC * NS
C = 32
T = 315
EP = T * NW * C
NP = 10240
RS = NP // NS
DR = NP // 128
TW = 128
BR = 1024


def _sc_edge_kernel(do):
    mesh = plsc.VectorSubcoreMesh(core_axis_name="c", subcore_axis_name="s")

    @functools.partial(
        pl.kernel,
        out_type=(
            jax.ShapeDtypeStruct((NC, NP, TW), jnp.float32),
            jax.ShapeDtypeStruct((NC, DR, 128), jnp.float32),
        ),
        mesh=mesh,
        compiler_params=pltpu.CompilerParams(needs_layout_passes=False),
        scratch_types=[
            pltpu.VMEM((C,), jnp.int32),
            pltpu.VMEM((C,), jnp.int32),
            pltpu.VMEM((C,), jnp.int32),
            pltpu.VMEM((C,), jnp.int32),
            pltpu.VMEM((C,), jnp.int32),
            pltpu.VMEM((C,), jnp.int32),
            pltpu.VMEM((C, TW), jnp.float32),
            pltpu.VMEM((C, TW), jnp.float32),
            pltpu.VMEM((C, TW), jnp.float32),
            pltpu.VMEM((C, TW), jnp.float32),
            pltpu.VMEM((C, TW), jnp.float32),
            pltpu.VMEM((C, TW), jnp.float32),
            pltpu.VMEM((C, TW), jnp.float32),
            pltpu.VMEM((C, TW), jnp.float32),
            pltpu.VMEM((C, TW), jnp.float32),
            pltpu.VMEM((do,), jnp.float32),
            pltpu.VMEM((DR, 128), jnp.float32),
            pltpu.VMEM((DR,), jnp.int32),
            pltpu.VMEM_SHARED((NP, TW), jnp.float32),
            pltpu.VMEM_SHARED((DR, 128), jnp.float32),
            pltpu.SemaphoreType.DMA,
            pltpu.SemaphoreType.DMA,
            pltpu.SemaphoreType.DMA,
            pltpu.SemaphoreType.DMA,
            pltpu.SemaphoreType.DMA,
            pltpu.SemaphoreType.DMA,
        ],
    )
    def edge_kernel(a_hbm, b_hbm, xl_hbm, src_hbm, dst_hbm, att_hbm, z_hbm,
                    num_out, den_out,
                    is0, is1, is2, id0, id1, id2, ra0, ra1, ra2,
                    rb0, rb1, rb2, rx0, rx1, rx2, att_v, den_local, row_ids,
                    num_sh, den_sh, g0, g1, g2, s0, s1, s2):
        idx_src = [is0, is1, is2]
        idx_dst = [id0, id1, id2]
        rows_a = [ra0, ra1, ra2]
        rows_b = [rb0, rb1, rb2]
        rows_xl = [rx0, rx1, rx2]
        gsem = [g0, g1, g2]
        ssem = [s0, s1, s2]

        c = lax.axis_index("c")
        s = lax.axis_index("s")
        wid = s * NC + c

        pltpu.sync_copy(z_hbm.at[pl.ds(s * RS, RS)],
                        num_sh.at[pl.ds(s * RS, RS)])

        @pl.when(s < 10)
        def _():
            pltpu.sync_copy(z_hbm.at[pl.ds(s * 8, 8)],
                            den_sh.at[pl.ds(s * 8, 8)])

        pltpu.sync_copy(att_hbm, att_v)

        def zden_body(r, carry):
            for k in range(8):
                den_local[r, pl.ds(k * 16, 16)] = jnp.zeros((16,), jnp.float32)
            return carry

        lax.fori_loop(0, DR, zden_body, 0)

        def rid_body(j, carry):
            row_ids[pl.ds(j * 16, 16)] = lax.iota(jnp.int32, 16) + j * 16
            return carry

        lax.fori_loop(0, DR // 16, rid_body, 0)
        plsc.subcore_barrier()

        def fire(a, b):
            base = (wid + a * NW) * C
            pltpu.sync_copy(src_hbm.at[pl.ds(base, C)], idx_src[b])
            pltpu.sync_copy(dst_hbm.at[pl.ds(base, C)], idx_dst[b])
            pltpu.async_copy(a_hbm.at[idx_src[b]], rows_a[b], gsem[b])
            pltpu.async_copy(b_hbm.at[idx_dst[b]], rows_b[b], gsem[b])
            pltpu.async_copy(xl_hbm.at[idx_src[b]], rows_xl[b], gsem[b])

        def wait_gathers(b):
            pltpu.make_async_copy(
                a_hbm.at[idx_src[b]], rows_a[b], gsem[b]).wait()
            pltpu.make_async_copy(
                b_hbm.at[idx_dst[b]], rows_b[b], gsem[b]).wait()
            pltpu.make_async_copy(
                xl_hbm.at[idx_src[b]], rows_xl[b], gsem[b]).wait()

        def fire_scatter(b):
            pltpu.async_copy(rows_xl[b], num_sh.at[idx_dst[b]],
                             ssem[b], add=True)

        def wait_scatter(b):
            pltpu.make_async_copy(
                rows_xl[b], num_sh.at[idx_dst[b]], ssem[b]).wait()

        def compute(b):
            iota16 = lax.iota(jnp.int32, 16)

            def group_body(g, carry2):
                ev = iota16 + g * 16

                def alpha_blk(k16, acc):
                    base = k16 * 16
                    for dk in range(16):
                        kv = (iota16 + (base + dk)) & (do - 1)
                        m = (plsc.load_gather(rows_a[b], [ev, kv])
                             + plsc.load_gather(rows_b[b], [ev, kv]))
                        lr = jnp.maximum(m, 0.2 * m)
                        acc = acc + lr * plsc.load_gather(att_v, [kv])
                    return acc

                acc = lax.fori_loop(0, do // 16, alpha_blk,
                                    jnp.zeros((16,), jnp.float32))
                exv = jnp.exp(acc)

                def scale_blk(k16, carry3):
                    base = k16 * 16
                    for dk in range(16):
                        kv = (iota16 + (base + dk)) & (do - 1)
                        v = plsc.load_gather(rows_xl[b], [ev, kv])
                        plsc.store_scatter(rows_xl[b], [ev, kv], v * exv)
                    return carry3

                lax.fori_loop(0, do // 16, scale_blk, 0)
                dstv = idx_dst[b][pl.ds(g * 16, 16)]
                plsc.addupdate_scatter(den_local, [dstv >> 7, dstv & 127], exv)
                return carry2

            lax.fori_loop(0, C // 16, group_body, 0)

        fire(0, 0)
        fire(1, 1)

        def pipe_body(t, carry):
            for b in range(3):
                a = t * 3 + b
                wait_gathers(b)
                nb = (b + 2) % 3

                @pl.when(a + 2 < T)
                def _():
                    @pl.when(a >= 1)
                    def _():
                        wait_scatter(nb)
                    fire(a + 2, nb)

                compute(b)
                fire_scatter(b)
            return carry

        lax.fori_loop(0, T // 3, pipe_body, 0)
        for b in range(3):
            wait_scatter(b)
        pltpu.sync_copy(den_local, den_sh.at[row_ids], add=True)
        plsc.subcore_barrier()

        pltpu.sync_copy(num_sh.at[pl.ds(s * RS, RS)],
                        num_out.at[c, pl.ds(s * RS, RS)])

        @pl.when(s < 10)
        def _():
            pltpu.sync_copy(den_sh.at[pl.ds(s * 8, 8)],
                            den_out.at[c, pl.ds(s * 8, 8)])

    return edge_kernel


_sc_edge_128 = _sc_edge_kernel(128)
_sc_edge_64 = _sc_edge_kernel(64)


def _tc_layer(first):
    def body(*refs):
        if first:
            (h_ref, pos_ref, wlh_ref, wap_ref, wrh_ref, wbp_ref,
             xle_ref, a_ref, b_ref) = refs
            h = h_ref[...]
        else:
            (np_ref, dp_ref, bias_ref, pos_ref, wlh_ref, wap_ref, wrh_ref,
             wbp_ref, xle_ref, a_ref, b_ref) = refs
            num = np_ref[0] + np_ref[1]
            den = dp_ref[0] + dp_ref[1]
            h = num / (den[:, None] + 1e-16) + bias_ref[...]
            h = jnp.where(h > 0, h, jnp.exp(jnp.minimum(h, 0.0)) - 1.0)
        p = pos_ref[...]
        xl = (jnp.dot(h, wlh_ref[...], preferred_element_type=jnp.float32)
              + jnp.dot(p, wap_ref[0], preferred_element_type=jnp.float32))
        pa = jnp.dot(p, wap_ref[1], preferred_element_type=jnp.float32)
        xr = (jnp.dot(h, wrh_ref[...], preferred_element_type=jnp.float32)
              + jnp.dot(p, wbp_ref[0], preferred_element_type=jnp.float32))
        xle_ref[...] = xl
        a_ref[...] = xl - pa
        b_ref[...] = xr

    return body


def _run_tc_layer(h_or_numP, denF, bias, pos, Wlh, Wap2, Wrh, Wbp2, first):
    grid = (NP // BR,)
    full = lambda shape: pl.BlockSpec(shape, lambda i: (0,) * len(shape))
    w_specs = [full((TW, TW)), full((2, 3, TW)), full((TW, TW)),
               full((1, 3, TW))]
    row = lambda w: pl.BlockSpec((BR, w), lambda i: (i, 0))
    if first:
        in_specs = [row(TW), row(3)] + w_specs
        args = (h_or_numP, pos, Wlh, Wap2, Wrh, Wbp2)
    else:
        in_specs = [
            pl.BlockSpec((NC, BR, TW), lambda i: (0, i, 0)),
            pl.BlockSpec((NC, BR), lambda i: (0, i)),
            full((TW,)),
            row(3),
        ] + w_specs
        args = (h_or_numP, denF, bias, pos, Wlh, Wap2, Wrh, Wbp2)
    out_shapes = [jax.ShapeDtypeStruct((NP, TW), jnp.float32)] * 3
    out_specs = [row(TW)] * 3
    return pl.pallas_call(
        _tc_layer(first),
        grid=grid,
        in_specs=in_specs,
        out_specs=out_specs,
        out_shape=out_shapes,
    )(*args)


def _tc_final_body(np_ref, dp_ref, bias_ref, out_ref):
    num = np_ref[0] + np_ref[1]
    den = dp_ref[0] + dp_ref[1]
    h = num[:, :64] / (den[:, None] + 1e-16) + bias_ref[...]
    out_ref[...] = jnp.where(h > 0, h, jnp.exp(jnp.minimum(h, 0.0)) - 1.0)


def _run_tc_final(numP, denF, bias):
    return pl.pallas_call(
        _tc_final_body,
        grid=(NP // BR,),
        in_specs=[
            pl.BlockSpec((NC, BR, TW), lambda i: (0, i, 0)),
            pl.BlockSpec((NC, BR), lambda i: (0, i)),
            pl.BlockSpec((64,), lambda i: (0,)),
        ],
        out_specs=pl.BlockSpec((BR, 64), lambda i: (i, 0)),
        out_shape=jax.ShapeDtypeStruct((NP, 64), jnp.float32),
    )(numP, denF, bias)


def _prep(Wl, Wr, We, do):
    Wlh, Wlp = Wl[:TW], Wl[TW:]
    Wrh, Wrp = Wr[:TW], Wr[TW:]
    if do < TW:
        pad = [(0, 0), (0, TW - do)]
        Wlh, Wlp = jnp.pad(Wlh, pad), jnp.pad(Wlp, pad)
        Wrh, Wrp = jnp.pad(Wrh, pad), jnp.pad(Wrp, pad)
        We = jnp.pad(We, pad)
    Wap2 = jnp.stack([Wlp, We])
    Wbp2 = (Wrp + We)[None]
    return Wlh, Wap2, Wrh, Wbp2


def kernel(x, edge_index, pos, Wl0, Wr0, We0, att0, b0, Wl1, Wr1, We1, att1,
           b1, Wl2, Wr2, We2, att2, b2):
    src = jnp.pad(edge_index[0], (0, EP - E), constant_values=N)
    dst = jnp.pad(edge_index[1], (0, EP - E), constant_values=N)
    xp = jnp.pad(x, [(0, NP - N), (0, 0)])
    posp = jnp.pad(pos, [(0, NP - N), (0, 0)])
    z = jnp.zeros((NP, TW), jnp.float32)

    Wlh, Wap2, Wrh, Wbp2 = _prep(Wl0, Wr0, We0, 128)
    xle, a, b = _run_tc_layer(xp, None, None, posp, Wlh, Wap2, Wrh, Wbp2, True)
    numP, denP = _sc_edge_128(a, b, xle, src, dst, att0, z)

    Wlh, Wap2, Wrh, Wbp2 = _prep(Wl1, Wr1, We1, 128)
    xle, a, b = _run_tc_layer(numP, denP.reshape(NC, NP), b0, posp, Wlh, Wap2,
                              Wrh, Wbp2, False)
    numP, denP = _sc_edge_128(a, b, xle, src, dst, att1, z)

    Wlh, Wap2, Wrh, Wbp2 = _prep(Wl2, Wr2, We2, 64)
    xle, a, b = _run_tc_layer(numP, denP.reshape(NC, NP), b1, posp, Wlh, Wap2,
                              Wrh, Wbp2, False)
    numP, denP = _sc_edge_64(a, b, xle, src, dst, att2, z)

    out = _run_tc_final(numP, denP.reshape(NC, NP), b2)
    return out[:N]


# speedup vs baseline: 3.4099x; 1.1680x over previous
import functools

import jax
import jax.numpy as jnp
from jax import lax
from jax.experimental import pallas as pl
from jax.experimental.pallas import tpu as pltpu
from jax.experimental.pallas import tpu_sc as plsc

N = 10000
E = 320000
NC = 2
NS = 16
NW = NC * NS
C = 32
T = 315
EP = T * NW * C
NP = 10240
RS = NP // NS
DR = NP // 128
TW = 128
BR = 1024


def _sc_edge_kernel(do):
    mesh = plsc.VectorSubcoreMesh(core_axis_name="c", subcore_axis_name="s")

    @functools.partial(
        pl.kernel,
        out_type=(
            jax.ShapeDtypeStruct((NC, NP, TW), jnp.float32),
            jax.ShapeDtypeStruct((NC, DR, 128), jnp.float32),
        ),
        mesh=mesh,
        compiler_params=pltpu.CompilerParams(needs_layout_passes=False),
        scratch_types=[
            pltpu.VMEM((C,), jnp.int32),
            pltpu.VMEM((C,), jnp.int32),
            pltpu.VMEM((C,), jnp.int32),
            pltpu.VMEM((C,), jnp.int32),
            pltpu.VMEM((C,), jnp.int32),
            pltpu.VMEM((C,), jnp.int32),
            pltpu.VMEM((C, TW), jnp.float32),
            pltpu.VMEM((C, TW), jnp.float32),
            pltpu.VMEM((C, TW), jnp.float32),
            pltpu.VMEM((C, TW), jnp.float32),
            pltpu.VMEM((C, TW), jnp.float32),
            pltpu.VMEM((C, TW), jnp.float32),
            pltpu.VMEM((C, TW), jnp.float32),
            pltpu.VMEM((C, TW), jnp.float32),
            pltpu.VMEM((C, TW), jnp.float32),
            pltpu.VMEM((do,), jnp.float32),
            pltpu.VMEM((DR, 128), jnp.float32),
            pltpu.VMEM((DR,), jnp.int32),
            pltpu.VMEM_SHARED((NP, TW), jnp.float32),
            pltpu.VMEM_SHARED((DR, 128), jnp.float32),
            pltpu.SemaphoreType.DMA,
            pltpu.SemaphoreType.DMA,
            pltpu.SemaphoreType.DMA,
            pltpu.SemaphoreType.DMA,
            pltpu.SemaphoreType.DMA,
            pltpu.SemaphoreType.DMA,
        ],
    )
    def edge_kernel(a_hbm, b_hbm, xl_hbm, src_hbm, dst_hbm, att_hbm, z_hbm,
                    num_out, den_out,
                    is0, is1, is2, id0, id1, id2, ra0, ra1, ra2,
                    rb0, rb1, rb2, rx0, rx1, rx2, att_v, den_local, row_ids,
                    num_sh, den_sh, g0, g1, g2, s0, s1, s2):
        idx_src = [is0, is1, is2]
        idx_dst = [id0, id1, id2]
        rows_a = [ra0, ra1, ra2]
        rows_b = [rb0, rb1, rb2]
        rows_xl = [rx0, rx1, rx2]
        gsem = [g0, g1, g2]
        ssem = [s0, s1, s2]

        c = lax.axis_index("c")
        s = lax.axis_index("s")
        wid = s * NC + c

        pltpu.sync_copy(z_hbm.at[pl.ds(s * RS, RS)],
                        num_sh.at[pl.ds(s * RS, RS)])

        @pl.when(s < 10)
        def _():
            pltpu.sync_copy(z_hbm.at[pl.ds(s * 8, 8)],
                            den_sh.at[pl.ds(s * 8, 8)])

        pltpu.sync_copy(att_hbm, att_v)

        def zden_body(r, carry):
            for k in range(8):
                den_local[r, pl.ds(k * 16, 16)] = jnp.zeros((16,), jnp.float32)
            return carry

        lax.fori_loop(0, DR, zden_body, 0)

        def rid_body(j, carry):
            row_ids[pl.ds(j * 16, 16)] = lax.iota(jnp.int32, 16) + j * 16
            return carry

        lax.fori_loop(0, DR // 16, rid_body, 0)
        plsc.subcore_barrier()

        def fire(a, b):
            base = (wid + a * NW) * C
            pltpu.sync_copy(src_hbm.at[pl.ds(base, C)], idx_src[b])
            pltpu.sync_copy(dst_hbm.at[pl.ds(base, C)], idx_dst[b])
            pltpu.async_copy(a_hbm.at[idx_src[b]], rows_a[b], gsem[b])
            pltpu.async_copy(b_hbm.at[idx_dst[b]], rows_b[b], gsem[b])
            pltpu.async_copy(xl_hbm.at[idx_src[b]], rows_xl[b], gsem[b])

        def wait_gathers(b):
            pltpu.make_async_copy(
                a_hbm.at[idx_src[b]], rows_a[b], gsem[b]).wait()
            pltpu.make_async_copy(
                b_hbm.at[idx_dst[b]], rows_b[b], gsem[b]).wait()
            pltpu.make_async_copy(
                xl_hbm.at[idx_src[b]], rows_xl[b], gsem[b]).wait()

        def fire_scatter(b):
            pltpu.async_copy(rows_xl[b], num_sh.at[idx_dst[b]],
                             ssem[b], add=True)

        def wait_scatter(b):
            pltpu.make_async_copy(
                rows_xl[b], num_sh.at[idx_dst[b]], ssem[b]).wait()

        def compute(b):
            iota16 = lax.iota(jnp.int32, 16)
            ev0 = iota16
            ev1 = iota16 + 16

            def alpha_blk(k16, accs):
                acc0, acc1 = accs
                base = k16 * 16
                for dk in range(16):
                    kv = (iota16 + (base + dk)) & (do - 1)
                    av = plsc.load_gather(att_v, [kv])
                    m0 = (plsc.load_gather(rows_a[b], [ev0, kv])
                          + plsc.load_gather(rows_b[b], [ev0, kv]))
                    m1 = (plsc.load_gather(rows_a[b], [ev1, kv])
                          + plsc.load_gather(rows_b[b], [ev1, kv]))
                    acc0 = acc0 + jnp.maximum(m0, 0.2 * m0) * av
                    acc1 = acc1 + jnp.maximum(m1, 0.2 * m1) * av
                return acc0, acc1

            z16 = jnp.zeros((16,), jnp.float32)
            acc0, acc1 = lax.fori_loop(0, do // 16, alpha_blk, (z16, z16))
            ex0 = jnp.exp(acc0)
            ex1 = jnp.exp(acc1)

            def scale_blk(k16, carry3):
                base = k16 * 16
                for dk in range(16):
                    kv = (iota16 + (base + dk)) & (do - 1)
                    v0 = plsc.load_gather(rows_xl[b], [ev0, kv])
                    v1 = plsc.load_gather(rows_xl[b], [ev1, kv])
                    plsc.store_scatter(rows_xl[b], [ev0, kv], v0 * ex0)
                    plsc.store_scatter(rows_xl[b], [ev1, kv], v1 * ex1)
                return carry3

            lax.fori_loop(0, do // 16, scale_blk, 0)
            d0 = idx_dst[b][pl.ds(0, 16)]
            d1 = idx_dst[b][pl.ds(16, 16)]
            plsc.addupdate_scatter(den_local, [d0 >> 7, d0 & 127], ex0)
            plsc.addupdate_scatter(den_local, [d1 >> 7, d1 & 127], ex1)

        fire(0, 0)
        fire(1, 1)

        def pipe_body(t, carry):
            for b in range(3):
                a = t * 3 + b
                wait_gathers(b)
                nb = (b + 2) % 3

                @pl.when(a + 2 < T)
                def _():
                    @pl.when(a >= 1)
                    def _():
                        wait_scatter(nb)
                    fire(a + 2, nb)

                compute(b)
                fire_scatter(b)
            return carry

        lax.fori_loop(0, T // 3, pipe_body, 0)
        for b in range(3):
            wait_scatter(b)
        pltpu.sync_copy(den_local, den_sh.at[row_ids], add=True)
        plsc.subcore_barrier()

        pltpu.sync_copy(num_sh.at[pl.ds(s * RS, RS)],
                        num_out.at[c, pl.ds(s * RS, RS)])

        @pl.when(s < 10)
        def _():
            pltpu.sync_copy(den_sh.at[pl.ds(s * 8, 8)],
                            den_out.at[c, pl.ds(s * 8, 8)])

    return edge_kernel


_sc_edge_128 = _sc_edge_kernel(128)
_sc_edge_64 = _sc_edge_kernel(64)


def _tc_layer(first):
    def body(*refs):
        if first:
            (h_ref, pos_ref, wlh_ref, wap_ref, wrh_ref, wbp_ref,
             xle_ref, a_ref, b_ref) = refs
            h = h_ref[...]
        else:
            (np_ref, dp_ref, bias_ref, pos_ref, wlh_ref, wap_ref, wrh_ref,
             wbp_ref, xle_ref, a_ref, b_ref) = refs
            num = np_ref[0] + np_ref[1]
            den = dp_ref[0] + dp_ref[1]
            h = num / (den[:, None] + 1e-16) + bias_ref[...]
            h = jnp.where(h > 0, h, jnp.exp(jnp.minimum(h, 0.0)) - 1.0)
        p = pos_ref[...]
        xl = (jnp.dot(h, wlh_ref[...], preferred_element_type=jnp.float32)
              + jnp.dot(p, wap_ref[0], preferred_element_type=jnp.float32))
        pa = jnp.dot(p, wap_ref[1], preferred_element_type=jnp.float32)
        xr = (jnp.dot(h, wrh_ref[...], preferred_element_type=jnp.float32)
              + jnp.dot(p, wbp_ref[0], preferred_element_type=jnp.float32))
        xle_ref[...] = xl
        a_ref[...] = xl - pa
        b_ref[...] = xr

    return body


def _run_tc_layer(h_or_numP, denF, bias, pos, Wlh, Wap2, Wrh, Wbp2, first):
    grid = (NP // BR,)
    full = lambda shape: pl.BlockSpec(shape, lambda i: (0,) * len(shape))
    w_specs = [full((TW, TW)), full((2, 3, TW)), full((TW, TW)),
               full((1, 3, TW))]
    row = lambda w: pl.BlockSpec((BR, w), lambda i: (i, 0))
    if first:
        in_specs = [row(TW), row(3)] + w_specs
        args = (h_or_numP, pos, Wlh, Wap2, Wrh, Wbp2)
    else:
        in_specs = [
            pl.BlockSpec((NC, BR, TW), lambda i: (0, i, 0)),
            pl.BlockSpec((NC, BR), lambda i: (0, i)),
            full((TW,)),
            row(3),
        ] + w_specs
        args = (h_or_numP, denF, bias, pos, Wlh, Wap2, Wrh, Wbp2)
    out_shapes = [jax.ShapeDtypeStruct((NP, TW), jnp.float32)] * 3
    out_specs = [row(TW)] * 3
    return pl.pallas_call(
        _tc_layer(first),
        grid=grid,
        in_specs=in_specs,
        out_specs=out_specs,
        out_shape=out_shapes,
    )(*args)


def _tc_final_body(np_ref, dp_ref, bias_ref, out_ref):
    num = np_ref[0] + np_ref[1]
    den = dp_ref[0] + dp_ref[1]
    h = num[:, :64] / (den[:, None] + 1e-16) + bias_ref[...]
    out_ref[...] = jnp.where(h > 0, h, jnp.exp(jnp.minimum(h, 0.0)) - 1.0)


def _run_tc_final(numP, denF, bias):
    return pl.pallas_call(
        _tc_final_body,
        grid=(NP // BR,),
        in_specs=[
            pl.BlockSpec((NC, BR, TW), lambda i: (0, i, 0)),
            pl.BlockSpec((NC, BR), lambda i: (0, i)),
            pl.BlockSpec((64,), lambda i: (0,)),
        ],
        out_specs=pl.BlockSpec((BR, 64), lambda i: (i, 0)),
        out_shape=jax.ShapeDtypeStruct((NP, 64), jnp.float32),
    )(numP, denF, bias)


def _prep(Wl, Wr, We, do):
    Wlh, Wlp = Wl[:TW], Wl[TW:]
    Wrh, Wrp = Wr[:TW], Wr[TW:]
    if do < TW:
        pad = [(0, 0), (0, TW - do)]
        Wlh, Wlp = jnp.pad(Wlh, pad), jnp.pad(Wlp, pad)
        Wrh, Wrp = jnp.pad(Wrh, pad), jnp.pad(Wrp, pad)
        We = jnp.pad(We, pad)
    Wap2 = jnp.stack([Wlp, We])
    Wbp2 = (Wrp + We)[None]
    return Wlh, Wap2, Wrh, Wbp2


def kernel(x, edge_index, pos, Wl0, Wr0, We0, att0, b0, Wl1, Wr1, We1, att1,
           b1, Wl2, Wr2, We2, att2, b2):
    src = jnp.pad(edge_index[0], (0, EP - E), constant_values=N)
    dst = jnp.pad(edge_index[1], (0, EP - E), constant_values=N)
    xp = jnp.pad(x, [(0, NP - N), (0, 0)])
    posp = jnp.pad(pos, [(0, NP - N), (0, 0)])
    z = jnp.zeros((NP, TW), jnp.float32)

    Wlh, Wap2, Wrh, Wbp2 = _prep(Wl0, Wr0, We0, 128)
    xle, a, b = _run_tc_layer(xp, None, None, posp, Wlh, Wap2, Wrh, Wbp2, True)
    numP, denP = _sc_edge_128(a, b, xle, src, dst, att0, z)

    Wlh, Wap2, Wrh, Wbp2 = _prep(Wl1, Wr1, We1, 128)
    xle, a, b = _run_tc_layer(numP, denP.reshape(NC, NP), b0, posp, Wlh, Wap2,
                              Wrh, Wbp2, False)
    numP, denP = _sc_edge_128(a, b, xle, src, dst, att1, z)

    Wlh, Wap2, Wrh, Wbp2 = _prep(Wl2, Wr2, We2, 64)
    xle, a, b = _run_tc_layer(numP, denP.reshape(NC, NP), b1, posp, Wlh, Wap2,
                              Wrh, Wbp2, False)
    numP, denP = _sc_edge_64(a, b, xle, src, dst, att2, z)

    out = _run_tc_final(numP, denP.reshape(NC, NP), b2)
    return out[:N]


# speedup vs baseline: 4.7630x; 1.3968x over previous
import functools

import jax
import jax.numpy as jnp
from jax import lax
from jax.experimental import pallas as pl
from jax.experimental.pallas import tpu as pltpu
from jax.experimental.pallas import tpu_sc as plsc

N = 10000
E = 320000
NC = 2
NS = 16
NW = NC * NS
C = 32
T = 315
EP = T * NW * C
NP = 10240
RS = NP // NS
DR = NP // 128
TW = 128
BR = 1024


def _sc_edge_kernel(do):
    mesh = plsc.VectorSubcoreMesh(core_axis_name="c", subcore_axis_name="s")

    @functools.partial(
        pl.kernel,
        out_type=(
            jax.ShapeDtypeStruct((NC, NP, TW), jnp.float32),
            jax.ShapeDtypeStruct((NC, DR, 128), jnp.float32),
        ),
        mesh=mesh,
        compiler_params=pltpu.CompilerParams(needs_layout_passes=False),
        scratch_types=[
            pltpu.VMEM((C,), jnp.int32),
            pltpu.VMEM((C,), jnp.int32),
            pltpu.VMEM((C,), jnp.int32),
            pltpu.VMEM((C,), jnp.int32),
            pltpu.VMEM((C,), jnp.int32),
            pltpu.VMEM((C,), jnp.int32),
            pltpu.VMEM((C, TW), jnp.float32),
            pltpu.VMEM((C, TW), jnp.float32),
            pltpu.VMEM((C, TW), jnp.float32),
            pltpu.VMEM((C, TW), jnp.float32),
            pltpu.VMEM((C, TW), jnp.float32),
            pltpu.VMEM((C, TW), jnp.float32),
            pltpu.VMEM((C, TW), jnp.float32),
            pltpu.VMEM((C, TW), jnp.float32),
            pltpu.VMEM((C, TW), jnp.float32),
            pltpu.VMEM((do,), jnp.float32),
            pltpu.VMEM((DR, 128), jnp.float32),
            pltpu.VMEM((DR,), jnp.int32),
            pltpu.VMEM_SHARED((NP, TW), jnp.float32),
            pltpu.VMEM_SHARED((DR, 128), jnp.float32),
            pltpu.VMEM((C,), jnp.int32),
            pltpu.VMEM((C,), jnp.int32),
            pltpu.VMEM((C,), jnp.int32),
            pltpu.SemaphoreType.DMA,
            pltpu.SemaphoreType.DMA,
            pltpu.SemaphoreType.DMA,
            pltpu.SemaphoreType.DMA,
            pltpu.SemaphoreType.DMA,
            pltpu.SemaphoreType.DMA,
            pltpu.SemaphoreType.DMA,
            pltpu.SemaphoreType.DMA,
            pltpu.SemaphoreType.DMA,
        ],
    )
    def edge_kernel(a_hbm, b_hbm, xl_hbm, src_hbm, dst_hbm, att_hbm, z_hbm,
                    num_out, den_out,
                    is0, is1, is2, id0, id1, id2, ra0, ra1, ra2,
                    rb0, rb1, rb2, rx0, rx1, rx2, att_v, den_local, row_ids,
                    num_sh, den_sh, sc0, sc1, sc2,
                    g0, g1, g2, s0, s1, s2, i0, i1, i2):
        idx_src = [is0, is1, is2]
        idx_dst = [id0, id1, id2]
        rows_a = [ra0, ra1, ra2]
        rows_b = [rb0, rb1, rb2]
        rows_xl = [rx0, rx1, rx2]
        sc_idx = [sc0, sc1, sc2]
        gsem = [g0, g1, g2]
        ssem = [s0, s1, s2]
        isem = [i0, i1, i2]

        c = lax.axis_index("c")
        s = lax.axis_index("s")
        wid = s * NC + c

        pltpu.sync_copy(z_hbm.at[pl.ds(s * RS, RS)],
                        num_sh.at[pl.ds(s * RS, RS)])

        @pl.when(s < 10)
        def _():
            pltpu.sync_copy(z_hbm.at[pl.ds(s * 8, 8)],
                            den_sh.at[pl.ds(s * 8, 8)])

        pltpu.sync_copy(att_hbm, att_v)

        def zden_body(r, carry):
            for k in range(8):
                den_local[r, pl.ds(k * 16, 16)] = jnp.zeros((16,), jnp.float32)
            return carry

        lax.fori_loop(0, DR, zden_body, 0)

        def rid_body(j, carry):
            row_ids[pl.ds(j * 16, 16)] = lax.iota(jnp.int32, 16) + j * 16
            return carry

        lax.fori_loop(0, DR // 16, rid_body, 0)
        plsc.subcore_barrier()

        def fire_idx(a, b):
            base = (wid + a * NW) * C
            pltpu.async_copy(src_hbm.at[pl.ds(base, C)], idx_src[b], isem[b])
            pltpu.async_copy(dst_hbm.at[pl.ds(base, C)], idx_dst[b], isem[b])

        def wait_idx(a, b):
            base = (wid + a * NW) * C
            pltpu.make_async_copy(
                src_hbm.at[pl.ds(base, C)], idx_src[b], isem[b]).wait()
            pltpu.make_async_copy(
                dst_hbm.at[pl.ds(base, C)], idx_dst[b], isem[b]).wait()

        def fire_rows(a, b):
            wait_idx(a, b)
            pltpu.async_copy(a_hbm.at[idx_src[b]], rows_a[b], gsem[b])
            pltpu.async_copy(b_hbm.at[idx_dst[b]], rows_b[b], gsem[b])
            pltpu.async_copy(xl_hbm.at[idx_src[b]], rows_xl[b], gsem[b])

        def wait_gathers(b):
            pltpu.make_async_copy(
                a_hbm.at[idx_src[b]], rows_a[b], gsem[b]).wait()
            pltpu.make_async_copy(
                b_hbm.at[idx_dst[b]], rows_b[b], gsem[b]).wait()
            pltpu.make_async_copy(
                xl_hbm.at[idx_src[b]], rows_xl[b], gsem[b]).wait()

        def fire_scatter(b):
            pltpu.async_copy(rows_xl[b], num_sh.at[sc_idx[b]],
                             ssem[b], add=True)

        def wait_scatter(b):
            pltpu.make_async_copy(
                rows_xl[b], num_sh.at[sc_idx[b]], ssem[b]).wait()

        def snapshot_idx(b):
            for j in range(C // 16):
                sl = pl.ds(j * 16, 16)
                sc_idx[b][sl] = idx_dst[b][sl]

        def compute(b):
            iota16 = lax.iota(jnp.int32, 16)
            ev0 = iota16
            ev1 = iota16 + 16

            def alpha_blk(k16, accs):
                acc0, acc1 = accs
                base = k16 * 16
                for dk in range(16):
                    kv = (iota16 + (base + dk)) & (do - 1)
                    av = plsc.load_gather(att_v, [kv])
                    m0 = (plsc.load_gather(rows_a[b], [ev0, kv])
                          + plsc.load_gather(rows_b[b], [ev0, kv]))
                    m1 = (plsc.load_gather(rows_a[b], [ev1, kv])
                          + plsc.load_gather(rows_b[b], [ev1, kv]))
                    acc0 = acc0 + jnp.maximum(m0, 0.2 * m0) * av
                    acc1 = acc1 + jnp.maximum(m1, 0.2 * m1) * av
                return acc0, acc1

            z16 = jnp.zeros((16,), jnp.float32)
            acc0, acc1 = lax.fori_loop(0, do // 16, alpha_blk, (z16, z16))
            ex0 = jnp.exp(acc0)
            ex1 = jnp.exp(acc1)

            def scale_blk(k16, carry3):
                base = k16 * 16
                for dk in range(16):
                    kv = (iota16 + (base + dk)) & (do - 1)
                    v0 = plsc.load_gather(rows_xl[b], [ev0, kv])
                    v1 = plsc.load_gather(rows_xl[b], [ev1, kv])
                    plsc.store_scatter(rows_xl[b], [ev0, kv], v0 * ex0)
                    plsc.store_scatter(rows_xl[b], [ev1, kv], v1 * ex1)
                return carry3

            lax.fori_loop(0, do // 16, scale_blk, 0)
            d0 = sc_idx[b][pl.ds(0, 16)]
            d1 = sc_idx[b][pl.ds(16, 16)]
            plsc.addupdate_scatter(den_local, [d0 >> 7, d0 & 127], ex0)
            plsc.addupdate_scatter(den_local, [d1 >> 7, d1 & 127], ex1)

        fire_idx(0, 0)
        fire_idx(1, 1)
        fire_idx(2, 2)
        fire_rows(0, 0)
        fire_rows(1, 1)

        def pipe_body(t, carry):
            for b in range(3):
                a = t * 3 + b
                wait_gathers(b)
                snapshot_idx(b)

                @pl.when(a + 3 < T)
                def _():
                    fire_idx(a + 3, b)

                nb = (b + 2) % 3

                @pl.when(a + 2 < T)
                def _():
                    @pl.when(a >= 1)
                    def _():
                        wait_scatter(nb)
                    fire_rows(a + 2, nb)

                compute(b)
                fire_scatter(b)
            return carry

        lax.fori_loop(0, T // 3, pipe_body, 0)
        for b in range(3):
            wait_scatter(b)
        pltpu.sync_copy(den_local, den_sh.at[row_ids], add=True)
        plsc.subcore_barrier()

        pltpu.sync_copy(num_sh.at[pl.ds(s * RS, RS)],
                        num_out.at[c, pl.ds(s * RS, RS)])

        @pl.when(s < 10)
        def _():
            pltpu.sync_copy(den_sh.at[pl.ds(s * 8, 8)],
                            den_out.at[c, pl.ds(s * 8, 8)])

    return edge_kernel


_sc_edge_128 = _sc_edge_kernel(128)
_sc_edge_64 = _sc_edge_kernel(64)


def _tc_layer(first):
    def body(*refs):
        if first:
            (h_ref, pos_ref, wlh_ref, wap_ref, wrh_ref, wbp_ref,
             xle_ref, a_ref, b_ref) = refs
            h = h_ref[...]
        else:
            (np_ref, dp_ref, bias_ref, pos_ref, wlh_ref, wap_ref, wrh_ref,
             wbp_ref, xle_ref, a_ref, b_ref) = refs
            num = np_ref[0] + np_ref[1]
            den = dp_ref[0] + dp_ref[1]
            h = num / (den[:, None] + 1e-16) + bias_ref[...]
            h = jnp.where(h > 0, h, jnp.exp(jnp.minimum(h, 0.0)) - 1.0)
        p = pos_ref[...]
        xl = (jnp.dot(h, wlh_ref[...], preferred_element_type=jnp.float32)
              + jnp.dot(p, wap_ref[0], preferred_element_type=jnp.float32))
        pa = jnp.dot(p, wap_ref[1], preferred_element_type=jnp.float32)
        xr = (jnp.dot(h, wrh_ref[...], preferred_element_type=jnp.float32)
              + jnp.dot(p, wbp_ref[0], preferred_element_type=jnp.float32))
        xle_ref[...] = xl
        a_ref[...] = xl - pa
        b_ref[...] = xr

    return body


def _run_tc_layer(h_or_numP, denF, bias, pos, Wlh, Wap2, Wrh, Wbp2, first):
    grid = (NP // BR,)
    full = lambda shape: pl.BlockSpec(shape, lambda i: (0,) * len(shape))
    w_specs = [full((TW, TW)), full((2, 3, TW)), full((TW, TW)),
               full((1, 3, TW))]
    row = lambda w: pl.BlockSpec((BR, w), lambda i: (i, 0))
    if first:
        in_specs = [row(TW), row(3)] + w_specs
        args = (h_or_numP, pos, Wlh, Wap2, Wrh, Wbp2)
    else:
        in_specs = [
            pl.BlockSpec((NC, BR, TW), lambda i: (0, i, 0)),
            pl.BlockSpec((NC, BR), lambda i: (0, i)),
            full((TW,)),
            row(3),
        ] + w_specs
        args = (h_or_numP, denF, bias, pos, Wlh, Wap2, Wrh, Wbp2)
    out_shapes = [jax.ShapeDtypeStruct((NP, TW), jnp.float32)] * 3
    out_specs = [row(TW)] * 3
    return pl.pallas_call(
        _tc_layer(first),
        grid=grid,
        in_specs=in_specs,
        out_specs=out_specs,
        out_shape=out_shapes,
    )(*args)


def _tc_final_body(np_ref, dp_ref, bias_ref, out_ref):
    num = np_ref[0] + np_ref[1]
    den = dp_ref[0] + dp_ref[1]
    h = num[:, :64] / (den[:, None] + 1e-16) + bias_ref[...]
    out_ref[...] = jnp.where(h > 0, h, jnp.exp(jnp.minimum(h, 0.0)) - 1.0)


def _run_tc_final(numP, denF, bias):
    return pl.pallas_call(
        _tc_final_body,
        grid=(NP // BR,),
        in_specs=[
            pl.BlockSpec((NC, BR, TW), lambda i: (0, i, 0)),
            pl.BlockSpec((NC, BR), lambda i: (0, i)),
            pl.BlockSpec((64,), lambda i: (0,)),
        ],
        out_specs=pl.BlockSpec((BR, 64), lambda i: (i, 0)),
        out_shape=jax.ShapeDtypeStruct((NP, 64), jnp.float32),
    )(numP, denF, bias)


def _prep(Wl, Wr, We, do):
    Wlh, Wlp = Wl[:TW], Wl[TW:]
    Wrh, Wrp = Wr[:TW], Wr[TW:]
    if do < TW:
        pad = [(0, 0), (0, TW - do)]
        Wlh, Wlp = jnp.pad(Wlh, pad), jnp.pad(Wlp, pad)
        Wrh, Wrp = jnp.pad(Wrh, pad), jnp.pad(Wrp, pad)
        We = jnp.pad(We, pad)
    Wap2 = jnp.stack([Wlp, We])
    Wbp2 = (Wrp + We)[None]
    return Wlh, Wap2, Wrh, Wbp2


def kernel(x, edge_index, pos, Wl0, Wr0, We0, att0, b0, Wl1, Wr1, We1, att1,
           b1, Wl2, Wr2, We2, att2, b2):
    src = jnp.pad(edge_index[0], (0, EP - E), constant_values=N)
    dst = jnp.pad(edge_index[1], (0, EP - E), constant_values=N)
    xp = jnp.pad(x, [(0, NP - N), (0, 0)])
    posp = jnp.pad(pos, [(0, NP - N), (0, 0)])
    z = jnp.zeros((NP, TW), jnp.float32)

    Wlh, Wap2, Wrh, Wbp2 = _prep(Wl0, Wr0, We0, 128)
    xle, a, b = _run_tc_layer(xp, None, None, posp, Wlh, Wap2, Wrh, Wbp2, True)
    numP, denP = _sc_edge_128(a, b, xle, src, dst, att0, z)

    Wlh, Wap2, Wrh, Wbp2 = _prep(Wl1, Wr1, We1, 128)
    xle, a, b = _run_tc_layer(numP, denP.reshape(NC, NP), b0, posp, Wlh, Wap2,
                              Wrh, Wbp2, False)
    numP, denP = _sc_edge_128(a, b, xle, src, dst, att1, z)

    Wlh, Wap2, Wrh, Wbp2 = _prep(Wl2, Wr2, We2, 64)
    xle, a, b = _run_tc_layer(numP, denP.reshape(NC, NP), b1, posp, Wlh, Wap2,
                              Wrh, Wbp2, False)
    numP, denP = _sc_edge_64(a, b, xle, src, dst, att2, z)

    out = _run_tc_final(numP, denP.reshape(NC, NP), b2)
    return out[:N]
